# baseline (speedup 1.0000x reference)
import functools

import jax
import jax.numpy as jnp
from jax import lax
from jax.experimental import pallas as pl
from jax.experimental.pallas import tpu as pltpu
from jax.experimental.pallas import tpu_sc as plsc

NUM_CORES = 2
NUM_SUBCORES = 16
NUM_WORKERS = NUM_CORES * NUM_SUBCORES
CHUNK = 128
NBUF = 4


@functools.lru_cache(maxsize=None)
def _make_gather(V, D, B):
    assert B % (NUM_WORKERS * CHUNK * NBUF) == 0
    b_per_w = B // NUM_WORKERS
    n_chunks = b_per_w // CHUNK
    n_outer = n_chunks // NBUF
    mesh = plsc.VectorSubcoreMesh(core_axis_name="c", subcore_axis_name="s")

    @functools.partial(
        pl.kernel,
        mesh=mesh,
        out_type=jax.ShapeDtypeStruct((B, D), jnp.float32),
        compiler_params=pltpu.CompilerParams(use_tc_tiling_on_sc=False),
        scratch_types=[
            pltpu.VMEM((NBUF, CHUNK), jnp.int32),
            pltpu.VMEM((NBUF, CHUNK, D), jnp.float32),
            pltpu.SemaphoreType.DMA((NBUF,)),
            pltpu.SemaphoreType.DMA((NBUF,)),
            pltpu.SemaphoreType.DMA((NBUF,)),
        ],
    )
    def gather_kernel(idx_hbm, table_hbm, out_hbm, idx_v, rows_v, isem, gsem, osem):
        wid = lax.axis_index("s") * NUM_CORES + lax.axis_index("c")
        base = wid * b_per_w
        limit = base + (n_chunks - 1) * CHUNK

        for b in range(NBUF):
            pltpu.async_copy(
                idx_hbm.at[pl.ds(base + b * CHUNK, CHUNK)], idx_v.at[b], isem.at[b]
            )

        def outer(go, carry):
            g0 = go * NBUF
            for b in range(NBUF):

                @pl.when(go > 0)
                def _():
                    pltpu.make_async_copy(
                        rows_v.at[b], out_hbm.at[pl.ds(base, CHUNK)], osem.at[b]
                    ).wait()

                pltpu.make_async_copy(
                    idx_hbm.at[pl.ds(base, CHUNK)], idx_v.at[b], isem.at[b]
                ).wait()
                pltpu.async_copy(table_hbm.at[idx_v.at[b]], rows_v.at[b], gsem.at[b])
            for b in range(NBUF):
                off = base + (g0 + b) * CHUNK
                pltpu.make_async_copy(
                    table_hbm.at[pl.ds(0, CHUNK)], rows_v.at[b], gsem.at[b]
                ).wait()
                pltpu.async_copy(rows_v.at[b], out_hbm.at[pl.ds(off, CHUNK)], osem.at[b])
                nxt = jnp.minimum(off + NBUF * CHUNK, limit)
                pltpu.async_copy(idx_hbm.at[pl.ds(nxt, CHUNK)], idx_v.at[b], isem.at[b])
            return carry

        lax.fori_loop(0, n_outer, outer, 0)
        for b in range(NBUF):
            pltpu.make_async_copy(
                rows_v.at[b], out_hbm.at[pl.ds(base, CHUNK)], osem.at[b]
            ).wait()
            pltpu.make_async_copy(
                idx_hbm.at[pl.ds(base, CHUNK)], idx_v.at[b], isem.at[b]
            ).wait()

    return gather_kernel


def kernel(input, table):
    B = input.size
    D = table.shape[1]
    idx_flat = input.reshape(B).astype(jnp.int32)
    out = _make_gather(table.shape[0], D, B)(idx_flat, table)
    return out.reshape(input.shape + (D,))


# speedup vs baseline: 1.0014x; 1.0014x over previous
import functools

import jax
import jax.numpy as jnp
from jax import lax
from jax.experimental import pallas as pl
from jax.experimental.pallas import tpu as pltpu
from jax.experimental.pallas import tpu_sc as plsc

NUM_CORES = 2
NUM_SUBCORES = 16
NUM_WORKERS = NUM_CORES * NUM_SUBCORES
NBUF = 4
SPLIT = (128, 72)


@functools.lru_cache(maxsize=None)
def _make_gather(V, D, B, T):
    assert B % (NUM_WORKERS * NBUF) == 0
    rows_per_w = B // NUM_WORKERS
    n_outer = rows_per_w // NBUF
    mesh = plsc.VectorSubcoreMesh(core_axis_name="c", subcore_axis_name="s")

    @functools.partial(
        pl.kernel,
        mesh=mesh,
        out_type=jax.ShapeDtypeStruct((B, T, D), jnp.float32),
        compiler_params=pltpu.CompilerParams(use_tc_tiling_on_sc=False),
        scratch_types=[
            pltpu.VMEM((NBUF, T), jnp.int32),
            pltpu.VMEM((NBUF, T, D), jnp.float32),
            pltpu.SemaphoreType.DMA((NBUF,)),
            pltpu.SemaphoreType.DMA((NBUF,)),
            pltpu.SemaphoreType.DMA((NBUF,)),
        ],
    )
    def gather_kernel(idx_hbm, table_hbm, out_hbm, idx_v, rows_v, isem, gsem, osem):
        wid = lax.axis_index("s") * NUM_CORES + lax.axis_index("c")
        base = wid * rows_per_w
        last = base + rows_per_w - NBUF

        for b in range(NBUF):
            pltpu.async_copy(idx_hbm.at[base + b], idx_v.at[b], isem.at[b])

        def outer(go, carry):
            r0 = base + go * NBUF
            for b in range(NBUF):

                @pl.when(go > 0)
                def _():
                    pltpu.make_async_copy(
                        rows_v.at[b], out_hbm.at[base], osem.at[b]
                    ).wait()

                pltpu.make_async_copy(idx_hbm.at[base], idx_v.at[b], isem.at[b]).wait()
                o = 0
                for w in SPLIT:
                    pltpu.async_copy(
                        table_hbm.at[idx_v.at[b, pl.ds(o, w)]],
                        rows_v.at[b, pl.ds(o, w)],
                        gsem.at[b],
                    )
                    o += w
            for b in range(NBUF):
                o = 0
                for w in SPLIT:
                    pltpu.make_async_copy(
                        table_hbm.at[pl.ds(0, w)],
                        rows_v.at[b, pl.ds(o, w)],
                        gsem.at[b],
                    ).wait()
                    o += w
                pltpu.async_copy(rows_v.at[b], out_hbm.at[r0 + b], osem.at[b])
                nxt = jnp.minimum(r0 + NBUF, last) + b
                pltpu.async_copy(idx_hbm.at[nxt], idx_v.at[b], isem.at[b])
            return carry

        lax.fori_loop(0, n_outer, outer, 0)
        for b in range(NBUF):
            pltpu.make_async_copy(rows_v.at[b], out_hbm.at[base], osem.at[b]).wait()
            pltpu.make_async_copy(idx_hbm.at[base], idx_v.at[b], isem.at[b]).wait()

    return gather_kernel


def kernel(input, table):
    B, T = input.shape
    D = table.shape[1]
    idx = input.astype(jnp.int32)
    return _make_gather(table.shape[0], D, B, T)(idx, table)
